# Initial kernel scaffold; baseline (speedup 1.0000x reference)
import jax
import jax.numpy as jnp
from jax.experimental import pallas as pl


def kernel(x, edge_index, edge_weight, W, b):
    raise NotImplementedError("write your pallas kernel here")


# speedup vs baseline: 3.6074x; 3.6074x over previous
import functools

import jax
import jax.numpy as jnp
from jax import lax
from jax.experimental import pallas as pl
from jax.experimental.pallas import tpu as pltpu
from jax.experimental.pallas import tpu_sc as plsc

N = 10000
E = 320000
D = 128

NC = 2
NS = 16
NW = NC * NS
EPW = E // NW
K = 80
NCHUNK = EPW // K
NPAD = 10240
RPT = NPAD // NS


def _matmul_body(x_ref, w_ref, o_ref):
    o_ref[...] = jnp.dot(x_ref[...], w_ref[...],
                         preferred_element_type=jnp.float32)


def _matmul(x, W):
    bm = 400
    return pl.pallas_call(
        _matmul_body,
        grid=(N // bm,),
        in_specs=[
            pl.BlockSpec((bm, D), lambda i: (i, 0)),
            pl.BlockSpec((D, D), lambda i: (0, 0)),
        ],
        out_specs=pl.BlockSpec((bm, D), lambda i: (i, 0)),
        out_shape=jax.ShapeDtypeStruct((N, D), jnp.float32),
    )(x, W)


def _spmm_kernel(support, row, col, w, out, acc, colv, rowv, wv, rows, sem):
    cid = lax.axis_index("c")
    sid = lax.axis_index("s")
    wid = sid * NC + cid

    zeros16 = jnp.zeros((16,), jnp.float32)

    def zrow(r, _):
        for j in range(D // 16):
            rows[r, pl.ds(16 * j, 16)] = zeros16
        return 0

    lax.fori_loop(0, K, zrow, 0)
    for z in range(RPT // K):
        pltpu.sync_copy(rows, acc.at[pl.ds(sid * RPT + z * K, K)])
    plsc.subcore_barrier()

    def chunk(i, _):
        base = wid * EPW + i * K
        pltpu.sync_copy(col.at[pl.ds(base, K)], colv)
        pltpu.sync_copy(row.at[pl.ds(base, K)], rowv)
        pltpu.sync_copy(w.at[pl.ds(base, K)], wv)
        pltpu.async_copy(support.at[colv], rows, sem).wait()

        def scale(g, _):
            wg = wv[pl.ds(16 * g, 16)]
            for e16 in range(16):
                wb = jnp.broadcast_to(wg[e16], (16,))
                e = 16 * g + e16
                for j in range(D // 16):
                    rows[e, pl.ds(16 * j, 16)] = rows[e, pl.ds(16 * j, 16)] * wb
            return 0

        lax.fori_loop(0, K // 16, scale, 0)
        pltpu.sync_copy(rows, acc.at[rowv], add=True)
        return 0

    lax.fori_loop(0, NCHUNK, chunk, 0)
    plsc.subcore_barrier()

    base = cid * NPAD + sid * RPT
    pltpu.sync_copy(acc.at[pl.ds(sid * RPT, RPT)], out.at[pl.ds(base, RPT)])


def _spmm_sc(support, row, col, w):
    mesh = plsc.VectorSubcoreMesh(core_axis_name="c", subcore_axis_name="s")
    return pl.kernel(
        _spmm_kernel,
        out_type=jax.ShapeDtypeStruct((NC * NPAD, D), jnp.float32),
        mesh=mesh,
        scratch_types=[
            pltpu.VMEM_SHARED((NPAD, D), jnp.float32),
            pltpu.VMEM((K,), jnp.int32),
            pltpu.VMEM((K,), jnp.int32),
            pltpu.VMEM((K,), jnp.float32),
            pltpu.VMEM((K, D), jnp.float32),
            pltpu.SemaphoreType.DMA,
        ],
    )(support, row, col, w)


def _combine_body(p0_ref, p1_ref, b_ref, o_ref):
    o_ref[...] = p0_ref[...] + p1_ref[...] + b_ref[...]


def _combine(partials, b2d):
    bm = 80
    return pl.pallas_call(
        _combine_body,
        grid=(N // bm,),
        in_specs=[
            pl.BlockSpec((bm, D), lambda i: (i, 0)),
            pl.BlockSpec((bm, D), lambda i: (i + NPAD // bm, 0)),
            pl.BlockSpec((1, D), lambda i: (0, 0)),
        ],
        out_specs=pl.BlockSpec((bm, D), lambda i: (i, 0)),
        out_shape=jax.ShapeDtypeStruct((N, D), jnp.float32),
    )(partials, partials, b2d)


@jax.jit
def kernel(x, edge_index, edge_weight, W, b):
    support = _matmul(x, W)
    partials = _spmm_sc(support, edge_index[0], edge_index[1], edge_weight)
    return _combine(partials, b.reshape(1, D))


# speedup vs baseline: 6.0332x; 1.6724x over previous
import jax
import jax.numpy as jnp
from jax import lax
from jax.experimental import pallas as pl
from jax.experimental.pallas import tpu as pltpu
from jax.experimental.pallas import tpu_sc as plsc

N = 10000
E = 320000
D = 128

NC = 2
NS = 16
NW = NC * NS
K = 128
NPAIR = 40
EPW = NPAIR * 2 * K
EPAD = NW * EPW
NPAD = 10240
RPT = NPAD // NS


def _matmul_body(x_ref, w_ref, o_ref):
    o_ref[...] = jnp.dot(x_ref[...], w_ref[...],
                         preferred_element_type=jnp.float32)


def _matmul(x, W):
    bm = 400
    return pl.pallas_call(
        _matmul_body,
        grid=(N // bm,),
        in_specs=[
            pl.BlockSpec((bm, D), lambda i: (i, 0)),
            pl.BlockSpec((D, D), lambda i: (0, 0)),
        ],
        out_specs=pl.BlockSpec((bm, D), lambda i: (i, 0)),
        out_shape=jax.ShapeDtypeStruct((N, D), jnp.float32),
    )(x, W)


def _spmm_kernel(support, row, col, w, out, acc,
                 colA, rowA, wA, colB, rowB, wB,
                 rows0, rows1, gsem0, gsem1, isemA, isemB):
    cid = lax.axis_index("c")
    sid = lax.axis_index("s")
    wid = sid * NC + cid

    zeros16 = jnp.zeros((16,), jnp.float32)

    def zrow(r, _):
        for j in range(D // 16):
            rows0[r, pl.ds(16 * j, 16)] = zeros16
        return 0

    lax.fori_loop(0, K, zrow, 0)
    for z in range(RPT // K):
        pltpu.sync_copy(rows0, acc.at[pl.ds(sid * RPT + z * K, K)])

    def load_set(pp, c, r, ww, sem):
        pltpu.async_copy(col.at[wid, pp], c, sem)
        pltpu.async_copy(row.at[wid, pp], r, sem)
        pltpu.async_copy(w.at[wid, pp], ww, sem)

    def wait_set(c, r, ww, sem):
        pltpu.make_async_copy(col, c, sem).wait()
        pltpu.make_async_copy(row, r, sem).wait()
        pltpu.make_async_copy(w, ww, sem).wait()

    def gather(cset, j, buf, gsem):
        pltpu.async_copy(support.at[cset.at[j]], buf, gsem)

    def gwait(buf, gsem):
        pltpu.make_async_copy(support, buf, gsem).wait()

    def scale_scatter(rset, wset, j, buf):
        def scale(g, _):
            wg = wset[j, pl.ds(16 * g, 16)]
            for e16 in range(16):
                wb = jnp.broadcast_to(wg[e16], (16,))
                e = 16 * g + e16
                for jj in range(D // 16):
                    buf[e, pl.ds(16 * jj, 16)] = (
                        buf[e, pl.ds(16 * jj, 16)] * wb)
            return 0

        lax.fori_loop(0, K // 16, scale, 0)
        pltpu.sync_copy(buf, acc.at[rset.at[j]], add=True)

    load_set(0, colA, rowA, wA, isemA)
    load_set(1, colB, rowB, wB, isemB)
    plsc.subcore_barrier()
    wait_set(colA, rowA, wA, isemA)
    gather(colA, 0, rows0, gsem0)

    def one_pair(pp, cX, rX, wX, isemX, cY, rY, wY, isemY):
        gather(cX, 1, rows1, gsem1)
        gwait(rows0, gsem0)
        scale_scatter(rX, wX, 0, rows0)
        gwait(rows1, gsem1)
        scale_scatter(rX, wX, 1, rows1)
        wait_set(cY, rY, wY, isemY)
        gather(cY, 0, rows0, gsem0)
        load_set(lax.rem(pp + 2, NPAIR), cX, rX, wX, isemX)

    def super_body(q, _):
        one_pair(2 * q, colA, rowA, wA, isemA, colB, rowB, wB, isemB)
        one_pair(2 * q + 1, colB, rowB, wB, isemB, colA, rowA, wA, isemA)
        return 0

    lax.fori_loop(0, NPAIR // 2, super_body, 0)
    gwait(rows0, gsem0)
    wait_set(colB, rowB, wB, isemB)

    plsc.subcore_barrier()

    base = cid * NPAD + sid * RPT
    pltpu.sync_copy(acc.at[pl.ds(sid * RPT, RPT)], out.at[pl.ds(base, RPT)])


def _spmm_sc(support, row4, col4, w4):
    mesh = plsc.VectorSubcoreMesh(core_axis_name="c", subcore_axis_name="s")
    return pl.kernel(
        _spmm_kernel,
        out_type=jax.ShapeDtypeStruct((NC * NPAD, D), jnp.float32),
        mesh=mesh,
        scratch_types=[
            pltpu.VMEM_SHARED((NPAD, D), jnp.float32),
            pltpu.VMEM((2, K), jnp.int32),
            pltpu.VMEM((2, K), jnp.int32),
            pltpu.VMEM((2, K), jnp.float32),
            pltpu.VMEM((2, K), jnp.int32),
            pltpu.VMEM((2, K), jnp.int32),
            pltpu.VMEM((2, K), jnp.float32),
            pltpu.VMEM((K, D), jnp.float32),
            pltpu.VMEM((K, D), jnp.float32),
            pltpu.SemaphoreType.DMA,
            pltpu.SemaphoreType.DMA,
            pltpu.SemaphoreType.DMA,
            pltpu.SemaphoreType.DMA,
        ],
    )(support, row4, col4, w4)


def _combine_body(p0_ref, p1_ref, b_ref, o_ref):
    o_ref[...] = p0_ref[...] + p1_ref[...] + b_ref[...]


def _combine(partials, b2d):
    bm = 80
    return pl.pallas_call(
        _combine_body,
        grid=(N // bm,),
        in_specs=[
            pl.BlockSpec((bm, D), lambda i: (i, 0)),
            pl.BlockSpec((bm, D), lambda i: (i + NPAD // bm, 0)),
            pl.BlockSpec((1, D), lambda i: (0, 0)),
        ],
        out_specs=pl.BlockSpec((bm, D), lambda i: (i, 0)),
        out_shape=jax.ShapeDtypeStruct((N, D), jnp.float32),
    )(partials, partials, b2d)


@jax.jit
def kernel(x, edge_index, edge_weight, W, b):
    support = _matmul(x, W)
    npadE = EPAD - E
    fill = jnp.arange(npadE, dtype=jnp.int32)
    row_p = jnp.concatenate([edge_index[0], N + fill % (NPAD - N)])
    col_p = jnp.concatenate([edge_index[1], fill % N])
    w_p = jnp.concatenate([edge_weight, jnp.zeros((npadE,), jnp.float32)])
    row4 = row_p.reshape(NW, NPAIR, 2, K)
    col4 = col_p.reshape(NW, NPAIR, 2, K)
    w4 = w_p.reshape(NW, NPAIR, 2, K)
    partials = _spmm_sc(support, row4, col4, w4)
    return _combine(partials, b.reshape(1, D))
